# baseline (speedup 1.0000x reference)
import numpy as np
import jax
import jax.numpy as jnp
from jax import lax
from jax.experimental import pallas as pl

_INP = 768
_TOPN = 4
_SZ = 14
_PAD = 1
_B = 32
_P = _SZ * _SZ
_NA = 6 * _P
_C1 = _INP // 2
_C2 = _INP // 4
_HP = _SZ + 2 * _PAD
_G = 8
_NG = _B // _G
_PP = 256
_NAP = 6 * _PP

_INTERPRET = False
_STAGE = 4
_CAST16 = True

_HI = lax.Precision.HIGHEST


def _make_anchors():
    stride = 1
    size = 3
    scales = [2 ** (1.0 / 3.0), 2 ** (2.0 / 3.0)]
    aspect_ratios = [0.667, 1, 1.5]
    out = np.zeros((0, 4), dtype=np.float32)
    oy = np.arange(0.5, 0.5 + stride * _SZ, stride).reshape(_SZ, 1)
    ox = np.arange(0.5, 0.5 + stride * _SZ, stride).reshape(1, _SZ)
    tmpl = np.zeros((_SZ, _SZ, 4), dtype=np.float32)
    tmpl[:, :, 0] = oy
    tmpl[:, :, 1] = ox
    for scale in scales:
        for ar in aspect_ratios:
            cam = tmpl.copy()
            cam[:, :, 2] = size * scale / float(ar) ** 0.5
            cam[:, :, 3] = size * scale * float(ar) ** 0.5
            eam = np.concatenate(
                (cam[..., :2] - cam[..., 2:4] / 2.0, cam[..., :2] + cam[..., 2:4] / 2.0),
                axis=-1)
            out = np.concatenate((out, eam.reshape(-1, 4)))
    return out


_EA = (_make_anchors() + 1).astype(np.int64)


def _pair_iou(anchors):
    a = anchors.astype(np.float32)
    start_max = np.maximum(a[:, None, 0:2], a[None, :, 0:2])
    end_min = np.minimum(a[:, None, 2:4], a[None, :, 2:4])
    lengths = end_min - start_max
    inter = lengths[..., 0] * lengths[..., 1]
    inter[np.logical_or(lengths[..., 0] < 0, lengths[..., 1] < 0)] = 0
    area = (a[:, 2] - a[:, 0]) * (a[:, 3] - a[:, 1])
    return inter / (area[:, None] + area[None, :] - inter)


_SUPP = (_pair_iou(_EA) >= 0.25).astype(np.float32)

_Y0 = np.clip(_EA[:, 0], 0, _HP - 1)
_X0 = np.clip(_EA[:, 1], 0, _HP - 1)
_Y1 = np.maximum(_Y0 + 1, np.minimum(_EA[:, 2], _HP))
_X1 = np.maximum(_X0 + 1, np.minimum(_EA[:, 3], _HP))
_WMAP = np.zeros((_NA, _P), dtype=np.float32)
for _a in range(_NA):
    _h = int(_Y1[_a] - _Y0[_a])
    _w = int(_X1[_a] - _X0[_a])
    _inv = 1.0 / float(_h * _w)
    for _i in range(int(_Y0[_a]), int(_Y1[_a])):
        _sy = min(max(_i - 1, 0), _SZ - 1)
        for _j in range(int(_X0[_a]), int(_X1[_a])):
            _sx = min(max(_j - 1, 0), _SZ - 1)
            _WMAP[_a, _sy * _SZ + _sx] += _inv
del _a, _h, _w, _inv, _i, _sy, _j, _sx

_OFFS = [(dy, dx) for dy in (-1, 0, 1) for dx in (-1, 0, 1)]
_MASKS = np.zeros((9, 1, _P), dtype=np.float32)
for _k, (_dy, _dx) in enumerate(_OFFS):
    for _pp in range(_P):
        _y, _x = _pp // _SZ, _pp % _SZ
        if 0 <= _y + _dy < _SZ and 0 <= _x + _dx < _SZ:
            _MASKS[_k, 0, _pp] = 1.0
del _k, _dy, _dx, _pp, _y, _x

_APAD = (np.arange(_NA) // _P) * _PP + (np.arange(_NA) % _P)
_SUPP_P = np.zeros((_NAP, _NAP), dtype=np.float32)
_SUPP_P[np.ix_(_APAD, _APAD)] = _SUPP
_WMAP_P = np.zeros((_NAP, _PP), dtype=np.float32)
_WMAP_P[_APAD, :_P] = _WMAP


def _conv1_stats_kernel(x_ref, w_ref, b_ref, y_ref, st_ref):
    pi = pl.program_id(0)

    @pl.when(pi == 0)
    def _init():
        st_ref[...] = jnp.zeros_like(st_ref)

    s1 = jnp.zeros((_C1, 1), jnp.float32)
    s2 = jnp.zeros((_C1, 1), jnp.float32)
    zpad = jnp.zeros((_C1, _PP - _P), jnp.float32)
    for i in range(_G):
        y = jnp.dot(w_ref[...], x_ref[i],
                    preferred_element_type=jnp.float32) + b_ref[...]
        y_ref[i] = jnp.concatenate([y, zpad], axis=1)
        s1 = s1 + jnp.sum(y, axis=1, keepdims=True)
        s2 = s2 + jnp.sum(y * y, axis=1, keepdims=True)
    st_ref[...] += jnp.concatenate([s1, s2], axis=1)


def _feat_scores_kernel(y_ref, st_ref, bnw_ref, bnb_ref, wd_ref, bd_ref,
                        wt_ref, bt_ref, mask_ref, f_ref, sc_ref):
    n = float(_B * _P)
    mean = st_ref[:, 0:1] / n
    var = st_ref[:, 1:2] / n - mean * mean
    scale = bnw_ref[...] / jnp.sqrt(var + 1e-5)
    shift = bnb_ref[...] - mean * scale

    mmdt = jnp.bfloat16 if _CAST16 else jnp.float32
    m16 = mask_ref[...].astype(mmdt)
    z = jnp.zeros((_C1, 16), mmdt)
    zpad = jnp.zeros((_C1, _PP - _P), jnp.float32)
    ninf = jnp.full((6, _PP - _P), -jnp.inf, jnp.float32)
    cols = []
    for i in range(_G):
        f = jnp.maximum(y_ref[i][:, :_P] * scale + shift, 0.0)
        f_ref[i] = jnp.concatenate([f, zpad], axis=1)
        f16 = f.astype(mmdt)
        fpad = jnp.concatenate([z, f16, z], axis=1)
        shifted = []
        for k, (dy, dx) in enumerate(_OFFS):
            o = dy * _SZ + dx
            shifted.append(fpad[:, 16 + o:16 + o + _P] * m16[k])
        cols.append(jnp.concatenate(shifted, axis=0))
    s_all = jnp.concatenate(cols, axis=1)
    acc = jnp.dot(wd_ref[...].astype(mmdt), s_all,
                  preferred_element_type=jnp.float32) + bd_ref[...]
    d16 = jnp.maximum(acc, 0.0).astype(mmdt)
    sc = jnp.dot(wt_ref[...].astype(mmdt), d16,
                 preferred_element_type=jnp.float32) + bt_ref[...]
    for i in range(_G):
        sc_ref[i] = jnp.concatenate([sc[:, i * _P:(i + 1) * _P], ninf], axis=1)


def _nms_kernel(sc_ref, supp_ref, wmap_ref, wsel_ref):
    scores = sc_ref[...]
    lane = lax.broadcasted_iota(jnp.int32, (_B, _NAP), 1)
    active = jnp.ones((_B, _NAP), jnp.float32)
    supp = supp_ref[...]
    wmap = wmap_ref[...]
    for t in range(_TOPN):
        masked = jnp.where(active > 0, scores, -jnp.inf)
        m = jnp.max(masked, axis=1, keepdims=True)
        selv = jnp.max(jnp.where(masked == m, lane, -1), axis=1, keepdims=True)
        oh = lane == selv
        rows = jnp.dot(oh.astype(supp.dtype), supp,
                       preferred_element_type=jnp.float32)
        active = active * (1.0 - rows)
        wsel_ref[:, t, :] = jnp.dot(oh.astype(jnp.float32), wmap,
                                    precision=_HI,
                                    preferred_element_type=jnp.float32)


def _crop_kernel(f_ref, wsel_ref, out_ref):
    for i in range(_G):
        out_ref[i] = lax.dot_general(
            wsel_ref[i], f_ref[i],
            dimension_numbers=(((1,), (1,)), ((), ())),
            precision=_HI,
            preferred_element_type=jnp.float32)


def kernel(x, conv1_w, conv1_b, bn_w, bn_b, down_w, down_b, tidy_w, tidy_b):
    if _STAGE == 0:
        return jnp.zeros((_B * _TOPN, _C1, 1, 1), jnp.float32) + x[0, 0, 0, 0]
    if _STAGE == -1:
        return jnp.zeros((_B * _TOPN, _C1, 1, 1), jnp.float32) + jnp.sum(x)
    if _STAGE == -2:
        return jnp.zeros((_B * _TOPN, _C1, 1, 1), jnp.float32) + jnp.sum(x.reshape(_B, _INP, _P)[:, :, 100])
    x3 = x.reshape(_B, _INP, _P)
    w1 = conv1_w.reshape(_C1, _INP)
    b1 = conv1_b.reshape(_C1, 1)

    y1, st = pl.pallas_call(
        _conv1_stats_kernel,
        grid=(_NG,),
        in_specs=[
            pl.BlockSpec((_G, _INP, _P), lambda b: (b, 0, 0)),
            pl.BlockSpec((_C1, _INP), lambda b: (0, 0)),
            pl.BlockSpec((_C1, 1), lambda b: (0, 0)),
        ],
        out_specs=[
            pl.BlockSpec((_G, _C1, _PP), lambda b: (b, 0, 0)),
            pl.BlockSpec((_C1, 2), lambda b: (0, 0)),
        ],
        out_shape=[
            jax.ShapeDtypeStruct((_B, _C1, _PP), jnp.float32),
            jax.ShapeDtypeStruct((_C1, 2), jnp.float32),
        ],
        interpret=_INTERPRET,
    )(x3, w1, b1)

    if _STAGE == 1:
        return jnp.zeros((_B * _TOPN, _C1, 1, 1), jnp.float32) + y1[0, 0, 0] + st[0, 0]

    wd = jnp.transpose(down_w, (0, 2, 3, 1)).reshape(_C2, 9 * _C1)
    bd = down_b.reshape(_C2, 1)
    wt = tidy_w.reshape(6, _C2)
    bt = tidy_b.reshape(6, 1)
    masks = jnp.asarray(_MASKS)

    feat, sc = pl.pallas_call(
        _feat_scores_kernel,
        grid=(_NG,),
        in_specs=[
            pl.BlockSpec((_G, _C1, _PP), lambda b: (b, 0, 0)),
            pl.BlockSpec((_C1, 2), lambda b: (0, 0)),
            pl.BlockSpec((_C1, 1), lambda b: (0, 0)),
            pl.BlockSpec((_C1, 1), lambda b: (0, 0)),
            pl.BlockSpec((_C2, 9 * _C1), lambda b: (0, 0)),
            pl.BlockSpec((_C2, 1), lambda b: (0, 0)),
            pl.BlockSpec((6, _C2), lambda b: (0, 0)),
            pl.BlockSpec((6, 1), lambda b: (0, 0)),
            pl.BlockSpec((9, 1, _P), lambda b: (0, 0, 0)),
        ],
        out_specs=[
            pl.BlockSpec((_G, _C1, _PP), lambda b: (b, 0, 0)),
            pl.BlockSpec((_G, 6, _PP), lambda b: (b, 0, 0)),
        ],
        out_shape=[
            jax.ShapeDtypeStruct((_B, _C1, _PP), jnp.float32),
            jax.ShapeDtypeStruct((_B, 6, _PP), jnp.float32),
        ],
        interpret=_INTERPRET,
    )(y1, st, bn_w.reshape(_C1, 1), bn_b.reshape(_C1, 1),
      wd, bd, wt, bt, masks)

    if _STAGE == 2:
        return jnp.zeros((_B * _TOPN, _C1, 1, 1), jnp.float32) + sc[0, 0, 0] + feat[0, 0, 0]

    scores2 = sc.reshape(_B, _NAP)
    supp = jnp.asarray(_SUPP_P).astype(jnp.bfloat16)
    wmap = jnp.asarray(_WMAP_P)

    wsel = pl.pallas_call(
        _nms_kernel,
        out_shape=jax.ShapeDtypeStruct((_B, _TOPN, _PP), jnp.float32),
        interpret=_INTERPRET,
    )(scores2, supp, wmap)

    if _STAGE == 3:
        return jnp.zeros((_B * _TOPN, _C1, 1, 1), jnp.float32) + wsel[0, 0, 0] + feat[0, 0, 0]

    out = pl.pallas_call(
        _crop_kernel,
        grid=(_NG,),
        in_specs=[
            pl.BlockSpec((_G, _C1, _PP), lambda b: (b, 0, 0)),
            pl.BlockSpec((_G, _TOPN, _PP), lambda b: (b, 0, 0)),
        ],
        out_specs=pl.BlockSpec((_G, _TOPN, _C1), lambda b: (b, 0, 0)),
        out_shape=jax.ShapeDtypeStruct((_B, _TOPN, _C1), jnp.float32),
        interpret=_INTERPRET,
    )(feat, wsel)

    return out.reshape(_B * _TOPN, _C1, 1, 1)


# speedup vs baseline: 1.0971x; 1.0971x over previous
import numpy as np
import jax
import jax.numpy as jnp
from jax import lax
from jax.experimental import pallas as pl

_INP = 768
_TOPN = 4
_SZ = 14
_PAD = 1
_B = 32
_P = _SZ * _SZ
_NA = 6 * _P
_C1 = _INP // 2
_C2 = _INP // 4
_HP = _SZ + 2 * _PAD
_G = 8
_NG = _B // _G
_PP = 256
_NAP = 6 * _PP

_INTERPRET = False
_STAGE = 4
_CAST16 = True
_FDT = jnp.bfloat16

_HI = lax.Precision.HIGHEST


def _make_anchors():
    stride = 1
    size = 3
    scales = [2 ** (1.0 / 3.0), 2 ** (2.0 / 3.0)]
    aspect_ratios = [0.667, 1, 1.5]
    out = np.zeros((0, 4), dtype=np.float32)
    oy = np.arange(0.5, 0.5 + stride * _SZ, stride).reshape(_SZ, 1)
    ox = np.arange(0.5, 0.5 + stride * _SZ, stride).reshape(1, _SZ)
    tmpl = np.zeros((_SZ, _SZ, 4), dtype=np.float32)
    tmpl[:, :, 0] = oy
    tmpl[:, :, 1] = ox
    for scale in scales:
        for ar in aspect_ratios:
            cam = tmpl.copy()
            cam[:, :, 2] = size * scale / float(ar) ** 0.5
            cam[:, :, 3] = size * scale * float(ar) ** 0.5
            eam = np.concatenate(
                (cam[..., :2] - cam[..., 2:4] / 2.0, cam[..., :2] + cam[..., 2:4] / 2.0),
                axis=-1)
            out = np.concatenate((out, eam.reshape(-1, 4)))
    return out


_EA = (_make_anchors() + 1).astype(np.int64)


def _pair_iou(anchors):
    a = anchors.astype(np.float32)
    start_max = np.maximum(a[:, None, 0:2], a[None, :, 0:2])
    end_min = np.minimum(a[:, None, 2:4], a[None, :, 2:4])
    lengths = end_min - start_max
    inter = lengths[..., 0] * lengths[..., 1]
    inter[np.logical_or(lengths[..., 0] < 0, lengths[..., 1] < 0)] = 0
    area = (a[:, 2] - a[:, 0]) * (a[:, 3] - a[:, 1])
    return inter / (area[:, None] + area[None, :] - inter)


_SUPP = (_pair_iou(_EA) >= 0.25).astype(np.float32)

_Y0 = np.clip(_EA[:, 0], 0, _HP - 1)
_X0 = np.clip(_EA[:, 1], 0, _HP - 1)
_Y1 = np.maximum(_Y0 + 1, np.minimum(_EA[:, 2], _HP))
_X1 = np.maximum(_X0 + 1, np.minimum(_EA[:, 3], _HP))
_WMAP = np.zeros((_NA, _P), dtype=np.float32)
for _a in range(_NA):
    _h = int(_Y1[_a] - _Y0[_a])
    _w = int(_X1[_a] - _X0[_a])
    _inv = 1.0 / float(_h * _w)
    for _i in range(int(_Y0[_a]), int(_Y1[_a])):
        _sy = min(max(_i - 1, 0), _SZ - 1)
        for _j in range(int(_X0[_a]), int(_X1[_a])):
            _sx = min(max(_j - 1, 0), _SZ - 1)
            _WMAP[_a, _sy * _SZ + _sx] += _inv
del _a, _h, _w, _inv, _i, _sy, _j, _sx

_OFFS = [(dy, dx) for dy in (-1, 0, 1) for dx in (-1, 0, 1)]
_MASKS = np.zeros((9, 1, _P), dtype=np.float32)
for _k, (_dy, _dx) in enumerate(_OFFS):
    for _pp in range(_P):
        _y, _x = _pp // _SZ, _pp % _SZ
        if 0 <= _y + _dy < _SZ and 0 <= _x + _dx < _SZ:
            _MASKS[_k, 0, _pp] = 1.0
del _k, _dy, _dx, _pp, _y, _x

_APAD = (np.arange(_NA) // _P) * _PP + (np.arange(_NA) % _P)
_SUPP_P = np.zeros((_NAP, _NAP), dtype=np.float32)
_SUPP_P[np.ix_(_APAD, _APAD)] = _SUPP
_WMAP_P = np.zeros((_NAP, _PP), dtype=np.float32)
_WMAP_P[_APAD, :_P] = _WMAP


from jax.experimental.pallas import tpu as pltpu


def _mega_kernel(x_ref, w1_ref, b1_ref, bnw_ref, bnb_ref, wd_ref, bd_ref,
                 wt_ref, bt_ref, mask_ref, supp_ref, wmap_ref, out_ref,
                 y1_scr, f_scr, sc_scr, st_scr):
    pi = pl.program_id(0)

    @pl.when(pi == 0)
    def _init():
        st_scr[...] = jnp.zeros_like(st_scr)

    @pl.when(pi < _NG)
    def _phase_a():
        s1 = jnp.zeros((_C1, 1), jnp.float32)
        s2 = jnp.zeros((_C1, 1), jnp.float32)
        zpad = jnp.zeros((_C1, _PP - _P), jnp.float32)
        for i in range(_G):
            y = jnp.dot(w1_ref[...], x_ref[i],
                        preferred_element_type=jnp.float32) + b1_ref[...]
            y1_scr[pl.ds(pi * _G + i, 1)] = jnp.concatenate(
                [y, zpad], axis=1)[None]
            s1 = s1 + jnp.sum(y, axis=1, keepdims=True)
            s2 = s2 + jnp.sum(y * y, axis=1, keepdims=True)
        st_scr[...] += jnp.concatenate([s1, s2], axis=1)

    @pl.when((pi >= _NG) & (pi < 2 * _NG))
    def _phase_b():
        n = float(_B * _P)
        mean = st_scr[:, 0:1] / n
        var = st_scr[:, 1:2] / n - mean * mean
        scale = bnw_ref[...] / jnp.sqrt(var + 1e-5)
        shift = bnb_ref[...] - mean * scale

        mmdt = jnp.bfloat16 if _CAST16 else jnp.float32
        m16 = mask_ref[...].astype(mmdt)
        z = jnp.zeros((_C1, 16), mmdt)
        ninf = jnp.full((1, _PP - _P), -jnp.inf, jnp.float32)
        cols = []
        base = (pi - _NG) * _G
        for i in range(_G):
            y = y1_scr[pl.ds(base + i, 1)][0][:, :_P]
            f = jnp.maximum(y * scale + shift, 0.0)
            f16 = f.astype(_FDT)
            f_scr[pl.ds(base + i, 1)] = jnp.concatenate(
                [f16, jnp.zeros((_C1, _PP - _P), _FDT)], axis=1)[None]
            fs = f16.astype(mmdt)
            fpad = jnp.concatenate([z, fs, z], axis=1)
            shifted = []
            for k, (dy, dx) in enumerate(_OFFS):
                o = dy * _SZ + dx
                shifted.append(fpad[:, 16 + o:16 + o + _P] * m16[k])
            cols.append(jnp.concatenate(shifted, axis=0))
        s_all = jnp.concatenate(cols, axis=1)
        acc = jnp.dot(wd_ref[...].astype(mmdt), s_all,
                      preferred_element_type=jnp.float32) + bd_ref[...]
        d16 = jnp.maximum(acc, 0.0).astype(mmdt)
        sc = jnp.dot(wt_ref[...].astype(mmdt), d16,
                     preferred_element_type=jnp.float32) + bt_ref[...]
        for i in range(_G):
            for ch in range(6):
                row = sc[ch:ch + 1, i * _P:(i + 1) * _P]
                sc_scr[pl.ds(base + i, 1), pl.ds(ch * _PP, _PP)] = (
                    jnp.concatenate([row, ninf], axis=1))

    @pl.when(pi == 2 * _NG)
    def _phase_c():
        scores = sc_scr[...]
        lane = lax.broadcasted_iota(jnp.int32, (_B, _NAP), 1)
        active = jnp.ones((_B, _NAP), jnp.float32)
        supp = supp_ref[...]
        wmap = wmap_ref[...]
        wsel = []
        for t in range(_TOPN):
            masked = jnp.where(active > 0, scores, -jnp.inf)
            m = jnp.max(masked, axis=1, keepdims=True)
            selv = jnp.max(jnp.where(masked == m, lane, -1), axis=1,
                           keepdims=True)
            oh = lane == selv
            rows = jnp.dot(oh.astype(supp.dtype), supp,
                           preferred_element_type=jnp.float32)
            active = active * (1.0 - rows)
            wsel.append(jnp.dot(oh.astype(jnp.float32), wmap, precision=_HI,
                                preferred_element_type=jnp.float32))
        wall = jnp.stack(wsel, axis=1)
        for b in range(_B):
            fb = f_scr[pl.ds(b, 1)][0].astype(jnp.float32)
            out_ref[b] = lax.dot_general(
                wall[b], fb,
                dimension_numbers=(((1,), (1,)), ((), ())),
                precision=_HI,
                preferred_element_type=jnp.float32)


def kernel(x, conv1_w, conv1_b, bn_w, bn_b, down_w, down_b, tidy_w, tidy_b):
    x3 = x.reshape(_B, _INP, _P)
    w1 = conv1_w.reshape(_C1, _INP)
    b1 = conv1_b.reshape(_C1, 1)
    wd = jnp.transpose(down_w, (0, 2, 3, 1)).reshape(_C2, 9 * _C1)
    bd = down_b.reshape(_C2, 1)
    wt = tidy_w.reshape(6, _C2)
    bt = tidy_b.reshape(6, 1)
    masks = jnp.asarray(_MASKS)
    supp = jnp.asarray(_SUPP_P).astype(jnp.bfloat16)
    wmap = jnp.asarray(_WMAP_P)

    cz = lambda b: (0, 0)
    out = pl.pallas_call(
        _mega_kernel,
        grid=(2 * _NG + 1,),
        in_specs=[
            pl.BlockSpec((_G, _INP, _P), lambda b: (jnp.minimum(b, _NG - 1), 0, 0)),
            pl.BlockSpec((_C1, _INP), cz),
            pl.BlockSpec((_C1, 1), cz),
            pl.BlockSpec((_C1, 1), cz),
            pl.BlockSpec((_C1, 1), cz),
            pl.BlockSpec((_C2, 9 * _C1), cz),
            pl.BlockSpec((_C2, 1), cz),
            pl.BlockSpec((6, _C2), cz),
            pl.BlockSpec((6, 1), cz),
            pl.BlockSpec((9, 1, _P), lambda b: (0, 0, 0)),
            pl.BlockSpec((_NAP, _NAP), cz),
            pl.BlockSpec((_NAP, _PP), cz),
        ],
        out_specs=pl.BlockSpec((_B, _TOPN, _C1), lambda b: (0, 0, 0)),
        out_shape=jax.ShapeDtypeStruct((_B, _TOPN, _C1), jnp.float32),
        scratch_shapes=[
            pltpu.VMEM((_B, _C1, _PP), jnp.float32),
            pltpu.VMEM((_B, _C1, _PP), _FDT),
            pltpu.VMEM((_B, _NAP), jnp.float32),
            pltpu.VMEM((_C1, 2), jnp.float32),
        ],
        interpret=_INTERPRET,
    )(x3, w1, b1, bn_w.reshape(_C1, 1), bn_b.reshape(_C1, 1),
      wd, bd, wt, bt, masks, supp, wmap)

    return out.reshape(_B * _TOPN, _C1, 1, 1)


# speedup vs baseline: 1.0995x; 1.0022x over previous
import numpy as np
import jax
import jax.numpy as jnp
from jax import lax
from jax.experimental import pallas as pl

_INP = 768
_TOPN = 4
_SZ = 14
_PAD = 1
_B = 32
_P = _SZ * _SZ
_NA = 6 * _P
_C1 = _INP // 2
_C2 = _INP // 4
_HP = _SZ + 2 * _PAD
_G = 8
_NG = _B // _G
_PP = 256
_NAP = 6 * _PP

_HI = lax.Precision.HIGHEST


def _make_anchors():
    stride = 1
    size = 3
    scales = [2 ** (1.0 / 3.0), 2 ** (2.0 / 3.0)]
    aspect_ratios = [0.667, 1, 1.5]
    out = np.zeros((0, 4), dtype=np.float32)
    oy = np.arange(0.5, 0.5 + stride * _SZ, stride).reshape(_SZ, 1)
    ox = np.arange(0.5, 0.5 + stride * _SZ, stride).reshape(1, _SZ)
    tmpl = np.zeros((_SZ, _SZ, 4), dtype=np.float32)
    tmpl[:, :, 0] = oy
    tmpl[:, :, 1] = ox
    for scale in scales:
        for ar in aspect_ratios:
            cam = tmpl.copy()
            cam[:, :, 2] = size * scale / float(ar) ** 0.5
            cam[:, :, 3] = size * scale * float(ar) ** 0.5
            eam = np.concatenate(
                (cam[..., :2] - cam[..., 2:4] / 2.0, cam[..., :2] + cam[..., 2:4] / 2.0),
                axis=-1)
            out = np.concatenate((out, eam.reshape(-1, 4)))
    return out


_EA = (_make_anchors() + 1).astype(np.int64)


def _pair_iou(anchors):
    a = anchors.astype(np.float32)
    start_max = np.maximum(a[:, None, 0:2], a[None, :, 0:2])
    end_min = np.minimum(a[:, None, 2:4], a[None, :, 2:4])
    lengths = end_min - start_max
    inter = lengths[..., 0] * lengths[..., 1]
    inter[np.logical_or(lengths[..., 0] < 0, lengths[..., 1] < 0)] = 0
    area = (a[:, 2] - a[:, 0]) * (a[:, 3] - a[:, 1])
    return inter / (area[:, None] + area[None, :] - inter)


_SUPP = (_pair_iou(_EA) >= 0.25).astype(np.float32)

_Y0 = np.clip(_EA[:, 0], 0, _HP - 1)
_X0 = np.clip(_EA[:, 1], 0, _HP - 1)
_Y1 = np.maximum(_Y0 + 1, np.minimum(_EA[:, 2], _HP))
_X1 = np.maximum(_X0 + 1, np.minimum(_EA[:, 3], _HP))
_WMAP = np.zeros((_NA, _P), dtype=np.float32)
for _a in range(_NA):
    _h = int(_Y1[_a] - _Y0[_a])
    _w = int(_X1[_a] - _X0[_a])
    _inv = 1.0 / float(_h * _w)
    for _i in range(int(_Y0[_a]), int(_Y1[_a])):
        _sy = min(max(_i - 1, 0), _SZ - 1)
        for _j in range(int(_X0[_a]), int(_X1[_a])):
            _sx = min(max(_j - 1, 0), _SZ - 1)
            _WMAP[_a, _sy * _SZ + _sx] += _inv
del _a, _h, _w, _inv, _i, _sy, _j, _sx

_OFFS = [(dy, dx) for dy in (-1, 0, 1) for dx in (-1, 0, 1)]
_MASKS = np.zeros((9, 1, _P), dtype=np.float32)
for _k, (_dy, _dx) in enumerate(_OFFS):
    for _pp in range(_P):
        _y, _x = _pp // _SZ, _pp % _SZ
        if 0 <= _y + _dy < _SZ and 0 <= _x + _dx < _SZ:
            _MASKS[_k, 0, _pp] = 1.0
del _k, _dy, _dx, _pp, _y, _x

_APAD = (np.arange(_NA) // _P) * _PP + (np.arange(_NA) % _P)
_SUPP_P = np.zeros((_NAP, _NAP), dtype=np.float32)
_SUPP_P[np.ix_(_APAD, _APAD)] = _SUPP
_WMAP_P = np.zeros((_NAP, _PP), dtype=np.float32)
_WMAP_P[_APAD, :_P] = _WMAP


from jax.experimental.pallas import tpu as pltpu


def _mega_kernel(x_ref, w1_ref, b1_ref, bnw_ref, bnb_ref, wd_ref, bd_ref,
                 wt_ref, bt_ref, mask_ref, supp_ref, wmap_ref, out_ref,
                 y1_scr, f_scr, sc_scr, st_scr):
    pi = pl.program_id(0)

    @pl.when(pi == 0)
    def _init():
        st_scr[...] = jnp.zeros_like(st_scr)

    @pl.when(pi < _NG)
    def _phase_a():
        s1 = jnp.zeros((_C1, 1), jnp.float32)
        s2 = jnp.zeros((_C1, 1), jnp.float32)
        zpad = jnp.zeros((_C1, _PP - _P), jnp.float32)
        for i in range(_G):
            y = jnp.dot(w1_ref[...], x_ref[i],
                        preferred_element_type=jnp.float32) + b1_ref[...]
            y1_scr[pl.ds(pi * _G + i, 1)] = jnp.concatenate(
                [y, zpad], axis=1)[None]
            s1 = s1 + jnp.sum(y, axis=1, keepdims=True)
            s2 = s2 + jnp.sum(y * y, axis=1, keepdims=True)
        st_scr[...] += jnp.concatenate([s1, s2], axis=1)

    @pl.when((pi >= _NG) & (pi < 2 * _NG))
    def _phase_b():
        n = float(_B * _P)
        mean = st_scr[:, 0:1] / n
        var = st_scr[:, 1:2] / n - mean * mean
        scale = bnw_ref[...] / jnp.sqrt(var + 1e-5)
        shift = bnb_ref[...] - mean * scale

        m16 = mask_ref[...].astype(jnp.bfloat16)
        z = jnp.zeros((_C1, 16), jnp.bfloat16)
        ninf = jnp.full((1, _PP - _P), -jnp.inf, jnp.float32)
        cols = []
        base = (pi - _NG) * _G
        for i in range(_G):
            y = y1_scr[pl.ds(base + i, 1)][0][:, :_P]
            f = jnp.maximum(y * scale + shift, 0.0)
            f16 = f.astype(jnp.bfloat16)
            f_scr[pl.ds(base + i, 1)] = jnp.concatenate(
                [f16, jnp.zeros((_C1, _PP - _P), jnp.bfloat16)], axis=1)[None]
            fs = f16
            fpad = jnp.concatenate([z, fs, z], axis=1)
            shifted = []
            for k, (dy, dx) in enumerate(_OFFS):
                o = dy * _SZ + dx
                shifted.append(fpad[:, 16 + o:16 + o + _P] * m16[k])
            cols.append(jnp.concatenate(shifted, axis=0))
        s_all = jnp.concatenate(cols, axis=1)
        acc = jnp.dot(wd_ref[...].astype(jnp.bfloat16), s_all,
                      preferred_element_type=jnp.float32) + bd_ref[...]
        d16 = jnp.maximum(acc, 0.0).astype(jnp.bfloat16)
        sc = jnp.dot(wt_ref[...].astype(jnp.bfloat16), d16,
                     preferred_element_type=jnp.float32) + bt_ref[...]
        for i in range(_G):
            for ch in range(6):
                row = sc[ch:ch + 1, i * _P:(i + 1) * _P]
                sc_scr[pl.ds(base + i, 1), pl.ds(ch * _PP, _PP)] = (
                    jnp.concatenate([row, ninf], axis=1))

    @pl.when(pi == 2 * _NG)
    def _phase_c():
        scores = sc_scr[...]
        lane = lax.broadcasted_iota(jnp.int32, (_B, _NAP), 1)
        active = jnp.ones((_B, _NAP), jnp.float32)
        supp = supp_ref[...]
        wmap = wmap_ref[...]
        wsel = []
        for t in range(_TOPN):
            masked = jnp.where(active > 0, scores, -jnp.inf)
            m = jnp.max(masked, axis=1, keepdims=True)
            selv = jnp.max(jnp.where(masked == m, lane, -1), axis=1,
                           keepdims=True)
            oh = lane == selv
            rows = jnp.dot(oh.astype(supp.dtype), supp,
                           preferred_element_type=jnp.float32)
            active = active * (1.0 - rows)
            wsel.append(jnp.dot(oh.astype(jnp.float32), wmap, precision=_HI,
                                preferred_element_type=jnp.float32))
        wall = jnp.stack(wsel, axis=1)
        for b in range(_B):
            fb = f_scr[pl.ds(b, 1)][0].astype(jnp.float32)
            out_ref[b] = lax.dot_general(
                wall[b], fb,
                dimension_numbers=(((1,), (1,)), ((), ())),
                precision=_HI,
                preferred_element_type=jnp.float32)


def kernel(x, conv1_w, conv1_b, bn_w, bn_b, down_w, down_b, tidy_w, tidy_b):
    x3 = x.reshape(_B, _INP, _P)
    w1 = conv1_w.reshape(_C1, _INP)
    b1 = conv1_b.reshape(_C1, 1)
    wd = jnp.transpose(down_w, (0, 2, 3, 1)).reshape(_C2, 9 * _C1)
    bd = down_b.reshape(_C2, 1)
    wt = tidy_w.reshape(6, _C2)
    bt = tidy_b.reshape(6, 1)
    masks = jnp.asarray(_MASKS)
    supp = jnp.asarray(_SUPP_P).astype(jnp.bfloat16)
    wmap = jnp.asarray(_WMAP_P)

    cz = lambda b: (0, 0)
    out = pl.pallas_call(
        _mega_kernel,
        grid=(2 * _NG + 1,),
        in_specs=[
            pl.BlockSpec((_G, _INP, _P), lambda b: (jnp.minimum(b, _NG - 1), 0, 0)),
            pl.BlockSpec((_C1, _INP), cz),
            pl.BlockSpec((_C1, 1), cz),
            pl.BlockSpec((_C1, 1), cz),
            pl.BlockSpec((_C1, 1), cz),
            pl.BlockSpec((_C2, 9 * _C1), cz),
            pl.BlockSpec((_C2, 1), cz),
            pl.BlockSpec((6, _C2), cz),
            pl.BlockSpec((6, 1), cz),
            pl.BlockSpec((9, 1, _P), lambda b: (0, 0, 0)),
            pl.BlockSpec((_NAP, _NAP), cz),
            pl.BlockSpec((_NAP, _PP), cz),
        ],
        out_specs=pl.BlockSpec((_B, _TOPN, _C1), lambda b: (0, 0, 0)),
        out_shape=jax.ShapeDtypeStruct((_B, _TOPN, _C1), jnp.float32),
        scratch_shapes=[
            pltpu.VMEM((_B, _C1, _PP), jnp.float32),
            pltpu.VMEM((_B, _C1, _PP), jnp.bfloat16),
            pltpu.VMEM((_B, _NAP), jnp.float32),
            pltpu.VMEM((_C1, 2), jnp.float32),
        ],
    )(x3, w1, b1, bn_w.reshape(_C1, 1), bn_b.reshape(_C1, 1),
      wd, bd, wt, bt, masks, supp, wmap)

    return out.reshape(_B * _TOPN, _C1, 1, 1)
